# baseline (speedup 1.0000x reference)
import math

import jax
import jax.numpy as jnp
from jax import lax
from jax.experimental import pallas as pl

TOPK = 50
NBITS = 22
INT_MIN = -(2 ** 31)
MASK31 = 0x7FFFFFFF


def _to_float(pat_signed):
    bits = jnp.where(pat_signed < 0, pat_signed ^ MASK31, pat_signed)
    return lax.bitcast_convert_type(bits, jnp.float32)


def _slab_kernel(kt_ref, vx_ref, q_ref, qo_ref, out_ref):
    s = jnp.dot(kt_ref[0], q_ref[0],
                preferred_element_type=jnp.float32)
    s = s / jnp.float32(math.sqrt(128.0))

    ones_row = jnp.ones((1, s.shape[0]), jnp.float32)

    def body(it, t_pat):
        bit = jnp.int32(31) - it
        cand_pat = t_pat | lax.shift_left(jnp.int32(1), bit)
        cand_f = _to_float(cand_pat ^ INT_MIN)
        ind = jnp.where(s >= cand_f, 1.0, 0.0)
        cnt = jnp.dot(ones_row, ind,
                      preferred_element_type=jnp.float32)
        return jnp.where(cnt >= float(TOPK), cand_pat, t_pat)

    t_pat = lax.fori_loop(0, NBITS, body, jnp.zeros((1, 576), jnp.int32))
    t_val = _to_float(t_pat ^ INT_MIN)

    ge = s >= t_val
    rowmax = jnp.max(s, axis=0, keepdims=True)
    e = jnp.where(ge, jnp.exp(s - rowmax), 0.0)
    cnt_ge = jnp.dot(ones_row, jnp.where(ge, 1.0, 0.0),
                     preferred_element_type=jnp.float32)

    prod = jnp.dot(vx_ref[0], e.astype(jnp.bfloat16),
                   preferred_element_type=jnp.float32)
    sum_e = prod[513:514, :]
    e_t = jnp.exp(t_val - rowmax)
    inv = 1.0 / (sum_e - (cnt_ge - float(TOPK)) * e_t)

    out_ref[0, 0, :512, :] = prod[:512, :] * inv
    out_ref[0, 0, 512:, :] = qo_ref[0] * (prod[512:513, :] * inv)


@jax.jit
def kernel(keys_bank, values_bank, mask_bank, q_in, q_out, h, w):
    obj_n, d_key, bank_n = keys_bank.shape
    bs, d_val, n = q_out.shape
    keys_t = jnp.transpose(keys_bank, (0, 2, 1)).astype(jnp.bfloat16)
    vx = jnp.concatenate(
        [values_bank, mask_bank,
         jnp.ones((obj_n, 1, bank_n), jnp.float32)],
        axis=1).astype(jnp.bfloat16)
    q_b = q_in.astype(jnp.bfloat16)

    grid = (obj_n, bs)
    out = pl.pallas_call(
        _slab_kernel,
        grid=grid,
        in_specs=[
            pl.BlockSpec((1, bank_n, d_key), lambda i, b: (i, 0, 0)),
            pl.BlockSpec((1, d_val + 2, bank_n), lambda i, b: (i, 0, 0)),
            pl.BlockSpec((1, d_key, n), lambda i, b: (b, 0, 0)),
            pl.BlockSpec((1, d_val, n), lambda i, b: (b, 0, 0)),
        ],
        out_specs=pl.BlockSpec((1, 1, 2 * d_val, n),
                               lambda i, b: (b, i, 0, 0)),
        out_shape=jax.ShapeDtypeStruct((bs, obj_n, 2 * d_val, n),
                                       jnp.float32),
    )(keys_t, vx, q_b, q_out)
    return out
